# Initial kernel scaffold; baseline (speedup 1.0000x reference)
import jax
import jax.numpy as jnp
from jax.experimental import pallas as pl


def kernel(x, weight):
    raise NotImplementedError("write your pallas kernel here")


# speedup vs baseline: 13.0523x; 13.0523x over previous
import functools

import jax
import jax.numpy as jnp
from jax import lax
from jax.experimental import pallas as pl
from jax.experimental.pallas import tpu as pltpu
from jax.experimental.pallas import tpu_sc as plsc

N = 4096
C = 1024
B = 4
N_PIECES = 20
L = 16
V = N // L
NC, NS = 2, 16
NW = NC * NS
ROWS = B * C
RPW = ROWS // NW
WPAD = 32


def _log2(n):
    return n.bit_length() - 1


WV = 16


def _sort_asc(a):
    s, _ = plsc.sort_key_val(a, a)
    return s


def _window_pass(colbuf):
    @plsc.parallel_loop(0, V // WV)
    def _(w):
        base = w * (WV * L)
        regs = [colbuf[pl.ds(base + i * L, L)] for i in range(WV)]
        regs = [_sort_asc(r) for r in regs]
        kv = 2
        while kv <= WV:
            for blk in range(0, WV, kv):
                for i in range(kv // 2):
                    va, vb = blk + i, blk + kv - 1 - i
                    a = regs[va] if kv == 2 else _sort_asc(regs[va])
                    b = regs[vb] if kv == 2 else _sort_asc(regs[vb])
                    b = lax.rev(b, (0,))
                    regs[va] = jnp.minimum(a, b)
                    regs[vb] = lax.rev(jnp.maximum(a, b), (0,))
            dv = kv // 4
            while dv >= 1:
                for hi in range(WV // (2 * dv)):
                    for lo in range(dv):
                        va = hi * 2 * dv + lo
                        vb = va + dv
                        a, b = regs[va], regs[vb]
                        regs[va] = jnp.minimum(a, b)
                        regs[vb] = jnp.maximum(a, b)
                dv //= 2
            kv *= 2
        for i in range(WV):
            colbuf[pl.ds(base + i * L, L)] = regs[i]


def _merge_level2(bufs, kv):
    kvh = kv // 2
    lh = _log2(kvh)

    @plsc.parallel_loop(0, V // 2, unroll=2)
    def _(p):
        blk = lax.shift_right_logical(p, lh)
        i = lax.bitwise_and(p, kvh - 1)
        va = lax.bitwise_or(lax.shift_left(blk, lh + 1), i)
        vb = lax.shift_left(blk, lh + 1) + (kv - 1) - i
        a_start = va * L
        b_start = vb * L
        for cb in bufs:
            a = _sort_asc(cb[pl.ds(a_start, L)])
            b = _sort_asc(cb[pl.ds(b_start, L)])
            b = lax.rev(b, (0,))
            cb[pl.ds(a_start, L)] = jnp.minimum(a, b)
            cb[pl.ds(b_start, L)] = lax.rev(jnp.maximum(a, b), (0,))

    dv = kv // 4
    while dv >= WV:
        ldv = _log2(dv)

        @plsc.parallel_loop(0, V // 2, unroll=4)
        def _(p, ldv=ldv, dv=dv):
            hi = lax.shift_right_logical(p, ldv)
            lo = lax.bitwise_and(p, dv - 1)
            va = lax.bitwise_or(lax.shift_left(hi, ldv + 1), lo)
            a_start = va * L
            b_start = a_start + dv * L
            for cb in bufs:
                a = cb[pl.ds(a_start, L)]
                b = cb[pl.ds(b_start, L)]
                cb[pl.ds(a_start, L)] = jnp.minimum(a, b)
                cb[pl.ds(b_start, L)] = jnp.maximum(a, b)

        dv //= 2

    top_dv = min(kv // 4, WV // 2)

    for cb in bufs:

        @plsc.parallel_loop(0, V // WV)
        def _(w, cb=cb):
            base = w * (WV * L)
            regs = [cb[pl.ds(base + i * L, L)] for i in range(WV)]
            dv = top_dv
            while dv >= 1:
                for hi in range(WV // (2 * dv)):
                    for lo in range(dv):
                        va = hi * 2 * dv + lo
                        vb = va + dv
                        a, b = regs[va], regs[vb]
                        regs[va] = jnp.minimum(a, b)
                        regs[vb] = jnp.maximum(a, b)
                dv //= 2
            for i in range(WV):
                cb[pl.ds(base + i * L, L)] = regs[i]


def _sort_dot2(bufs, wtbuf, j0):
    for cb in bufs:
        _window_pass(cb)
    kv = 2 * WV
    while kv <= V:
        _merge_level2(bufs, kv)
        kv *= 2

    scale = jnp.float32(N_PIECES / (N - 1))
    iota = lax.iota(jnp.int32, L)
    woff0 = j0 * WPAD
    woff1 = woff0 + WPAD
    zero = jnp.zeros((L,), jnp.float32)

    @plsc.parallel_loop(0, V, unroll=2, carry=(zero, zero))
    def accs(v, carry):
        acc0, acc1 = carry
        rr = (N - 1) - (v * L + iota)
        pos = rr.astype(jnp.float32) * scale
        idx = pos.astype(jnp.int32)
        frac = pos - idx.astype(jnp.float32)
        idx1 = jnp.minimum(idx + 1, N_PIECES)
        s0 = _sort_asc(bufs[0][pl.ds(v * L, L)])
        wl0 = plsc.load_gather(wtbuf, [woff0 + idx])
        wr0 = plsc.load_gather(wtbuf, [woff0 + idx1])
        acc0 = acc0 + s0 * ((1.0 - frac) * wl0 + frac * wr0)
        s1 = _sort_asc(bufs[1][pl.ds(v * L, L)])
        wl1 = plsc.load_gather(wtbuf, [woff1 + idx])
        wr1 = plsc.load_gather(wtbuf, [woff1 + idx1])
        acc1 = acc1 + s1 * ((1.0 - frac) * wl1 + frac * wr1)
        return acc0, acc1

    return jnp.sum(accs[0]), jnp.sum(accs[1])


def _tec_body(xt_hbm, wt_hbm, out_hbm, b0, b1, b2, b3, wtbuf, outbuf,
              s0_, s1_, s2_, s3_):
    wid = lax.axis_index("s") * NC + lax.axis_index("c")
    base = wid * RPW
    c0 = lax.rem(base, C)
    pltpu.sync_copy(wt_hbm.at[pl.ds(c0 * WPAD, RPW * WPAD)], wtbuf)
    lane = lax.iota(jnp.int32, L)
    pair0 = ((b0, b1), (s0_, s1_))
    pair1 = ((b2, b3), (s2_, s3_))

    def start(rows, pair):
        bufs, sems = pair
        pltpu.async_copy(xt_hbm.at[rows[0]], bufs[0], sems[0])
        pltpu.async_copy(xt_hbm.at[rows[1]], bufs[1], sems[1])

    def wait(pair):
        bufs, sems = pair
        pltpu.make_async_copy(xt_hbm.at[base], bufs[0], sems[0]).wait()
        pltpu.make_async_copy(xt_hbm.at[base], bufs[1], sems[1]).wait()

    start((base, base + 1), pair0)

    def group(g, carry):
        def quad(k4, resvec):
            j0 = g * L + 4 * k4
            row0 = base + j0
            wait(pair0)
            start((row0 + 2, row0 + 3), pair1)
            sa, sb = _sort_dot2(pair0[0], wtbuf, j0)
            resvec = jnp.where(lane == 4 * k4, sa, resvec)
            resvec = jnp.where(lane == 4 * k4 + 1, sb, resvec)
            wait(pair1)
            start((jnp.minimum(row0 + 4, ROWS - 1),
                   jnp.minimum(row0 + 5, ROWS - 1)), pair0)
            sc, sd = _sort_dot2(pair1[0], wtbuf, j0 + 2)
            resvec = jnp.where(lane == 4 * k4 + 2, sc, resvec)
            return jnp.where(lane == 4 * k4 + 3, sd, resvec)

        resvec = lax.fori_loop(0, L // 4, quad, jnp.zeros((L,), jnp.float32))
        outbuf[pl.ds(g * L, L)] = resvec
        return carry

    lax.fori_loop(0, RPW // L, group, 0)
    wait(pair0)
    pltpu.sync_copy(outbuf, out_hbm.at[pl.ds(base, RPW)])


@functools.partial(
    pl.kernel,
    out_type=jax.ShapeDtypeStruct((ROWS,), jnp.float32),
    mesh=plsc.VectorSubcoreMesh(
        core_axis_name="c", subcore_axis_name="s",
        num_cores=NC, num_subcores=NS),
    scratch_types=[
        pltpu.VMEM((N,), jnp.float32),
        pltpu.VMEM((N,), jnp.float32),
        pltpu.VMEM((N,), jnp.float32),
        pltpu.VMEM((N,), jnp.float32),
        pltpu.VMEM((RPW * WPAD,), jnp.float32),
        pltpu.VMEM((RPW,), jnp.float32),
        pltpu.SemaphoreType.DMA,
        pltpu.SemaphoreType.DMA,
        pltpu.SemaphoreType.DMA,
        pltpu.SemaphoreType.DMA,
    ],
    compiler_params=pltpu.CompilerParams(needs_layout_passes=False),
)
def _fspool_sc(xt_hbm, wt_hbm, out_hbm, b0, b1, b2, b3, wtbuf, outbuf,
               s0_, s1_, s2_, s3_):
    _tec_body(xt_hbm, wt_hbm, out_hbm, b0, b1, b2, b3, wtbuf, outbuf,
              s0_, s1_, s2_, s3_)


def kernel(x, weight):
    b, n, c = x.shape
    xt = jnp.swapaxes(x, 1, 2).reshape(b * c, n)
    wt = jnp.zeros((c, WPAD), weight.dtype).at[:, : N_PIECES + 1].set(weight.T)
    out_flat = _fspool_sc(xt, wt.reshape(-1))
    return out_flat.reshape(b, c)
